# Initial kernel scaffold; baseline (speedup 1.0000x reference)
import jax
import jax.numpy as jnp
from jax.experimental import pallas as pl


def kernel(x, codebooks):
    raise NotImplementedError("write your pallas kernel here")


# speedup vs baseline: 1.3378x; 1.3378x over previous
import functools

import jax
import jax.numpy as jnp
from jax.experimental import pallas as pl

NUM_Q = 4
K = 1024
D = 256
BETA = 0.25
BN = 512


def _rvq_kernel(x_ref, cb_ref, xq_ref, loss_ref, idx_ref, *, n_total):
    i = pl.program_id(0)

    r = x_ref[...]
    rn = jnp.sum(r * r, axis=1, keepdims=True)
    xq_acc = jnp.zeros_like(r)
    loss_acc = jnp.zeros((), jnp.float32)
    idx_cols = []
    for s in range(NUM_Q):
        cb = cb_ref[s]
        cbn = jnp.sum(cb * cb, axis=1)
        scores = jnp.dot(r, cb.T, preferred_element_type=jnp.float32)
        d = rn + cbn[None, :] - 2.0 * scores
        m = jnp.min(d, axis=1, keepdims=True)
        iota = jax.lax.broadcasted_iota(jnp.int32, d.shape, 1)
        idx = jnp.min(jnp.where(d == m, iota, K), axis=1, keepdims=True)
        onehot = (iota == idx).astype(jnp.bfloat16)
        c1 = cb.astype(jnp.bfloat16)
        rem = cb - c1.astype(jnp.float32)
        c2 = rem.astype(jnp.bfloat16)
        c3 = (rem - c2.astype(jnp.float32)).astype(jnp.bfloat16)
        dot = lambda a, b: jnp.dot(a, b, preferred_element_type=jnp.float32)
        xq = (dot(onehot, c1) + dot(onehot, c2)) + dot(onehot, c3)
        t = xq - r
        x_res = r + t
        loss_acc = loss_acc + jnp.sum(t * t)
        r = r - x_res
        rn = jnp.sum(r * r, axis=1, keepdims=True)
        xq_acc = xq_acc + x_res
        idx_cols.append(idx)

    xq_ref[...] = xq_acc
    idx_ref[...] = jnp.concatenate(idx_cols, axis=1)

    scale = (1.0 + BETA) / (NUM_Q * n_total * D)

    @pl.when(i == 0)
    def _():
        loss_ref[...] = jnp.zeros((1, 1), jnp.float32)

    loss_ref[...] += (loss_acc * scale)[None, None]


def kernel(x, codebooks):
    n = x.shape[0]
    nb = n // BN
    xq, loss, idx = pl.pallas_call(
        functools.partial(_rvq_kernel, n_total=n),
        grid=(nb,),
        in_specs=[
            pl.BlockSpec((BN, D), lambda i: (i, 0)),
            pl.BlockSpec((NUM_Q, K, D), lambda i: (0, 0, 0)),
        ],
        out_specs=[
            pl.BlockSpec((BN, D), lambda i: (i, 0)),
            pl.BlockSpec((1, 1), lambda i: (0, 0)),
            pl.BlockSpec((BN, NUM_Q), lambda i: (i, 0)),
        ],
        out_shape=[
            jax.ShapeDtypeStruct((n, D), jnp.float32),
            jax.ShapeDtypeStruct((1, 1), jnp.float32),
            jax.ShapeDtypeStruct((n, NUM_Q), jnp.int32),
        ],
    )(x, codebooks)
    return xq, loss[0, 0], idx
